# baseline (speedup 1.0000x reference)
import jax
import jax.numpy as jnp
from jax import lax
from jax.experimental import pallas as pl
from jax.experimental.pallas import tpu as pltpu
from jax.experimental.pallas import tpu_sc as plsc

_BONE_A = (0, 0, 1, 2, 5, 5, 7, 6, 8, 5, 6, 11, 11, 13, 12, 14)
_BONE_B = (1, 2, 3, 4, 6, 7, 9, 8, 10, 11, 12, 12, 13, 15, 14, 16)
_B, _NP, _NJ, _ND, _NB = 16, 20, 17, 64, 16
_NJP = 24
_L = 16
_NG = _ND // _L
_NPT = _NP // 2
_IMW, _IMH = 1920.0, 1080.0


def _spl(s):
    return jnp.full((_L,), s, dtype=jnp.float32)


def _sqrt16(x):
    i = lax.bitcast_convert_type(x, jnp.int32)
    i = jnp.int32(0x5F3759DF) - (i >> 1)
    y = lax.bitcast_convert_type(i, jnp.float32)
    xh = x * jnp.float32(0.5)
    for _ in range(2):
        y = y * (jnp.float32(1.5) - xh * y * y)
    return x * y


def _sc_body(cams_hbm, x3_hbm, y3_hbm, z3_hbm, xr_hbm, yr_hbm, vis_hbm,
             score_hbm, sbl_hbm, bound_hbm, bound2_hbm,
             cams_v, xr_v, yr_v, vis_v, x3_v, y3_v, z3_v,
             xt_v, yt_v, mx_v, my_v, bi_v,
             score_v, sbl_v, bound_v, bound2_v, sem_in, sem_out):
    wid = lax.axis_index("s") * 2 + lax.axis_index("c")
    b = wid // 2
    pt0 = (wid % 2) * _NPT

    c1 = pltpu.async_copy(cams_hbm.at[b], cams_v, sem_in)
    c2 = pltpu.async_copy(xr_hbm.at[b], xr_v, sem_in)
    c3 = pltpu.async_copy(yr_hbm.at[b], yr_v, sem_in)
    c4 = pltpu.async_copy(vis_hbm.at[b], vis_v, sem_in)
    c1.wait(); c2.wait(); c3.wait(); c4.wait()

    fxv = cams_v[0]
    fyv = cams_v[1]
    cxv = cams_v[2]
    cyv = cams_v[3]
    nprv = cams_v[4]

    pltpu.async_copy(x3_hbm.at[b, pt0], x3_v.at[pl.ds(0, _NJ)], sem_in)
    pltpu.async_copy(y3_hbm.at[b, pt0], y3_v.at[pl.ds(0, _NJ)], sem_in)
    pltpu.async_copy(z3_hbm.at[b, pt0], z3_v.at[pl.ds(0, _NJ)], sem_in)

    def pt_body(pti, _):
        p = pti % 2
        pj = p * _NJ
        qj = (1 - p) * _NJ
        pk = p * _NB
        pt = pt0 + pti
        pltpu.make_async_copy(x3_hbm.at[b, pt], x3_v.at[pl.ds(pj, _NJ)], sem_in).wait()
        pltpu.make_async_copy(y3_hbm.at[b, pt], y3_v.at[pl.ds(pj, _NJ)], sem_in).wait()
        pltpu.make_async_copy(z3_hbm.at[b, pt], z3_v.at[pl.ds(pj, _NJ)], sem_in).wait()

        @pl.when(pti < _NPT - 1)
        def _prefetch():
            pltpu.async_copy(x3_hbm.at[b, pt + 1], x3_v.at[pl.ds(qj, _NJ)], sem_in)
            pltpu.async_copy(y3_hbm.at[b, pt + 1], y3_v.at[pl.ds(qj, _NJ)], sem_in)
            pltpu.async_copy(z3_hbm.at[b, pt + 1], z3_v.at[pl.ds(qj, _NJ)], sem_in)

        @pl.when(pti >= 2)
        def _drain_out():
            ptp = pt - 2
            pltpu.make_async_copy(score_v.at[pl.ds(pj, _NJ)], score_hbm.at[b, ptp],
                                  sem_out).wait()
            pltpu.make_async_copy(sbl_v.at[pl.ds(pk, _NB)], sbl_hbm.at[b, ptp],
                                  sem_out).wait()
            pltpu.make_async_copy(bound_v.at[pl.ds(pj, _NJ)], bound_hbm.at[b, ptp],
                                  sem_out).wait()
            pltpu.make_async_copy(bound2_v.at[pl.ds(pk, _NB)], bound2_hbm.at[b, ptp],
                                  sem_out).wait()

        @plsc.parallel_loop(0, _NJ)
        def _proj(j):
            for g in range(_NG):
                sl = pl.ds(g * _L, _L)
                z = jnp.maximum(z3_v[pj + j, sl], jnp.float32(1e-3))
                xt_v[j, sl] = x3_v[pj + j, sl] / z * fxv + cxv
                yt_v[j, sl] = y3_v[pj + j, sl] / z * fyv + cyv

        init = tuple(jnp.full((_L,), jnp.float32(3.0e38)) for _ in range(_NG)) \
            + tuple(jnp.zeros((_L,), jnp.int32) for _ in range(_NG))

        @plsc.parallel_loop(0, _NP, carry=init)
        def res(pr, carry):
            bds = list(carry[:_NG])
            bis = list(carry[_NG:])
            pr16 = jnp.full((_L,), pr, jnp.int32)
            base = pr16 * _NJP
            accs = [jnp.zeros((_L,), jnp.float32) for _ in range(_NG)]
            den = _spl(jnp.float32(1e-8))
            for j in range(_NJ):
                idxj = base + j
                xrs = plsc.load_gather(xr_v, [idxj])
                yrs = plsc.load_gather(yr_v, [idxj])
                vs = plsc.load_gather(vis_v, [idxj])
                den = den + vs
                for g in range(_NG):
                    sl = pl.ds(g * _L, _L)
                    dx = xt_v[j, sl] - xrs
                    dy = yt_v[j, sl] - yrs
                    accs[g] = accs[g] + vs * (dx * dx + dy * dy)
            valid = pr16.astype(jnp.float32) < nprv
            for g in range(_NG):
                d = accs[g] / den
                d = jnp.where(valid, d, _spl(jnp.float32(1e5)))
                take = d < bds[g]
                bis[g] = jnp.where(take, pr16, bis[g])
                bds[g] = jnp.where(take, d, bds[g])
            return tuple(bds) + tuple(bis)

        for g in range(_NG):
            bi_v[g] = res[_NG + g]

        @plsc.parallel_loop(0, _NG)
        def _g(g):
            sl = pl.ds(g * _L, _L)
            g17 = g * _NJ
            bibase = bi_v[g] * _NJP
            mv0 = None
            for j in range(_NJ):
                idxj = bibase + j
                mx = plsc.load_gather(xr_v, [idxj])
                my = plsc.load_gather(yr_v, [idxj])
                mv = plsc.load_gather(vis_v, [idxj])
                mx_v[g17 + j] = mx
                my_v[g17 + j] = my
                xtj = xt_v[j, sl]
                ytj = yt_v[j, sl]
                ddx = xtj - mx
                ddy = ytj - my
                md = ddx * ddx + ddy * ddy + jnp.float32(1e-12)
                score_v[pj + j, sl] = jnp.exp(_sqrt16(md) * jnp.float32(-0.02))
                inb = ((xtj >= jnp.float32(0.0)) & (ytj >= jnp.float32(0.0))
                       & (xtj <= jnp.float32(_IMW - 1))
                       & (ytj <= jnp.float32(_IMH - 1)))
                bound_v[pj + j, sl] = jnp.where(inb, mv, _spl(jnp.float32(0.0)))
                if j == 0:
                    mv0 = mv

            for k in range(_NB):
                a, c = _BONE_A[k], _BONE_B[k]
                ex = xt_v[a, sl] - xt_v[c, sl]
                ey = yt_v[a, sl] - yt_v[c, sl]
                blt = _sqrt16(ex * ex + ey * ey + jnp.float32(1e-12))
                exr = mx_v[g17 + a] - mx_v[g17 + c]
                eyr = my_v[g17 + a] - my_v[g17 + c]
                blr = _sqrt16(exr * exr + eyr * eyr + jnp.float32(1e-12))
                sbl_v[pk + k, sl] = jnp.exp(jnp.abs(blr - blt)
                                          * jnp.float32(-0.2))
                bound2_v[pk + k, sl] = mv0

        pltpu.async_copy(score_v.at[pl.ds(pj, _NJ)], score_hbm.at[b, pt], sem_out)
        pltpu.async_copy(sbl_v.at[pl.ds(pk, _NB)], sbl_hbm.at[b, pt], sem_out)
        pltpu.async_copy(bound_v.at[pl.ds(pj, _NJ)], bound_hbm.at[b, pt], sem_out)
        pltpu.async_copy(bound2_v.at[pl.ds(pk, _NB)], bound2_hbm.at[b, pt], sem_out)
        return 0

    lax.fori_loop(0, _NPT, pt_body, 0, unroll=False)

    for q in range(2):
        ptq = pt0 + _NPT - 2 + q
        pq = ptq % 2
        pltpu.make_async_copy(score_v.at[pl.ds(pq * _NJ, _NJ)], score_hbm.at[b, ptq],
                              sem_out).wait()
        pltpu.make_async_copy(sbl_v.at[pl.ds(pq * _NB, _NB)], sbl_hbm.at[b, ptq],
                              sem_out).wait()
        pltpu.make_async_copy(bound_v.at[pl.ds(pq * _NJ, _NJ)], bound_hbm.at[b, ptq],
                              sem_out).wait()
        pltpu.make_async_copy(bound2_v.at[pl.ds(pq * _NB, _NB)], bound2_hbm.at[b, ptq],
                              sem_out).wait()


@jax.jit
def kernel(poses_3d, poses_2d_ref, vis_ref, cam_f, cam_c, num_persons_ref):
    f32 = jnp.float32
    x3 = poses_3d[..., 0]
    y3 = poses_3d[..., 1]
    z3 = poses_3d[..., 2]
    pad = ((0, 0), (0, 0), (0, _NJP - _NJ))
    xr = jnp.pad(poses_2d_ref[..., 0], pad).reshape(_B, _NP * _NJP)
    yr = jnp.pad(poses_2d_ref[..., 1], pad).reshape(_B, _NP * _NJP)
    vis = jnp.pad(vis_ref, pad).reshape(_B, _NP * _NJP)
    cams = jnp.stack([cam_f[:, 0], cam_f[:, 1], cam_c[:, 0], cam_c[:, 1],
                      num_persons_ref.astype(f32),
                      jnp.zeros((_B,), f32), jnp.zeros((_B,), f32),
                      jnp.zeros((_B,), f32)], axis=1)
    cams16 = jnp.broadcast_to(cams[:, :, None], (_B, 8, _L)) + 0.0

    mesh = plsc.VectorSubcoreMesh(core_axis_name="c", subcore_axis_name="s",
                                  num_cores=2, num_subcores=16)
    out_type = [
        jax.ShapeDtypeStruct((_B, _NP, _NJ, _ND), f32),
        jax.ShapeDtypeStruct((_B, _NP, _NB, _ND), f32),
        jax.ShapeDtypeStruct((_B, _NP, _NJ, _ND), f32),
        jax.ShapeDtypeStruct((_B, _NP, _NB, _ND), f32),
    ]
    scratch = [
        pltpu.VMEM((8, _L), f32),
        pltpu.VMEM((_NP * _NJP,), f32),
        pltpu.VMEM((_NP * _NJP,), f32),
        pltpu.VMEM((_NP * _NJP,), f32),
        pltpu.VMEM((2 * _NJ, _ND), f32),
        pltpu.VMEM((2 * _NJ, _ND), f32),
        pltpu.VMEM((2 * _NJ, _ND), f32),
        pltpu.VMEM((_NJ, _ND), f32),
        pltpu.VMEM((_NJ, _ND), f32),
        pltpu.VMEM((_NG * _NJ, _L), f32),
        pltpu.VMEM((_NG * _NJ, _L), f32),
        pltpu.VMEM((_NG, _L), jnp.int32),
        pltpu.VMEM((2 * _NJ, _ND), f32),
        pltpu.VMEM((2 * _NB, _ND), f32),
        pltpu.VMEM((2 * _NJ, _ND), f32),
        pltpu.VMEM((2 * _NB, _ND), f32),
        pltpu.SemaphoreType.DMA,
        pltpu.SemaphoreType.DMA,
    ]
    outs = pl.kernel(
        _sc_body,
        out_type=out_type,
        mesh=mesh,
        scratch_types=scratch,
        compiler_params=pltpu.CompilerParams(needs_layout_passes=False),
    )(cams16, x3, y3, z3, xr, yr, vis)
    return tuple(outs)


# speedup vs baseline: 1.5018x; 1.5018x over previous
import jax
import jax.numpy as jnp
from jax import lax
from jax.experimental import pallas as pl
from jax.experimental.pallas import tpu as pltpu
from jax.experimental.pallas import tpu_sc as plsc

_BONE_A = (0, 0, 1, 2, 5, 5, 7, 6, 8, 5, 6, 11, 11, 13, 12, 14)
_BONE_B = (1, 2, 3, 4, 6, 7, 9, 8, 10, 11, 12, 12, 13, 15, 14, 16)
_B, _NP, _NJ, _ND, _NB = 16, 20, 17, 64, 16
_NJP = 24
_L = 16
_NG = _ND // _L
_NPT = _NP // 2
_IMW, _IMH = 1920.0, 1080.0


def _spl(s):
    return jnp.full((_L,), s, dtype=jnp.float32)


def _sqrt16(x):
    i = lax.bitcast_convert_type(x, jnp.int32)
    i = jnp.int32(0x5F3759DF) - (i >> 1)
    y = lax.bitcast_convert_type(i, jnp.float32)
    xh = x * jnp.float32(0.5)
    for _ in range(2):
        y = y * (jnp.float32(1.5) - xh * y * y)
    return x * y


def _sc_body(cams_hbm, x3_hbm, y3_hbm, z3_hbm, xr_hbm, yr_hbm, vis_hbm,
             score_hbm, sbl_hbm, bound_hbm, bound2_hbm,
             cams_v, xr_v, yr_v, vis_v, x3_v, y3_v, z3_v,
             xt_v, yt_v, mx_v, my_v, bi_v,
             score_v, sbl_v, bound_v, bound2_v, sem_in, sem_out):
    wid = lax.axis_index("s") * 2 + lax.axis_index("c")
    b = wid // 2
    pt0 = (wid % 2) * _NPT

    c1 = pltpu.async_copy(cams_hbm.at[b], cams_v, sem_in)
    c2 = pltpu.async_copy(xr_hbm.at[b], xr_v, sem_in)
    c3 = pltpu.async_copy(yr_hbm.at[b], yr_v, sem_in)
    c4 = pltpu.async_copy(vis_hbm.at[b], vis_v, sem_in)
    c1.wait(); c2.wait(); c3.wait(); c4.wait()

    fxv = cams_v[0]
    fyv = cams_v[1]
    cxv = cams_v[2]
    cyv = cams_v[3]
    nprv = cams_v[4]

    pltpu.async_copy(x3_hbm.at[b, pt0], x3_v.at[pl.ds(0, _NJ)], sem_in)
    pltpu.async_copy(y3_hbm.at[b, pt0], y3_v.at[pl.ds(0, _NJ)], sem_in)
    pltpu.async_copy(z3_hbm.at[b, pt0], z3_v.at[pl.ds(0, _NJ)], sem_in)

    def pt_body(pti, _):
        p = pti % 2
        pj = p * _NJ
        qj = (1 - p) * _NJ
        pk = p * _NB
        pt = pt0 + pti
        pltpu.make_async_copy(x3_hbm.at[b, pt], x3_v.at[pl.ds(pj, _NJ)], sem_in).wait()
        pltpu.make_async_copy(y3_hbm.at[b, pt], y3_v.at[pl.ds(pj, _NJ)], sem_in).wait()
        pltpu.make_async_copy(z3_hbm.at[b, pt], z3_v.at[pl.ds(pj, _NJ)], sem_in).wait()

        @pl.when(pti < _NPT - 1)
        def _prefetch():
            pltpu.async_copy(x3_hbm.at[b, pt + 1], x3_v.at[pl.ds(qj, _NJ)], sem_in)
            pltpu.async_copy(y3_hbm.at[b, pt + 1], y3_v.at[pl.ds(qj, _NJ)], sem_in)
            pltpu.async_copy(z3_hbm.at[b, pt + 1], z3_v.at[pl.ds(qj, _NJ)], sem_in)

        @pl.when(pti >= 2)
        def _drain_out():
            ptp = pt - 2
            pltpu.make_async_copy(score_v.at[pl.ds(pj, _NJ)], score_hbm.at[b, ptp],
                                  sem_out).wait()
            pltpu.make_async_copy(sbl_v.at[pl.ds(pk, _NB)], sbl_hbm.at[b, ptp],
                                  sem_out).wait()
            pltpu.make_async_copy(bound_v.at[pl.ds(pj, _NJ)], bound_hbm.at[b, ptp],
                                  sem_out).wait()
            pltpu.make_async_copy(bound2_v.at[pl.ds(pk, _NB)], bound2_hbm.at[b, ptp],
                                  sem_out).wait()

        @plsc.parallel_loop(0, _NJ)
        def _proj(j):
            for g in range(_NG):
                sl = pl.ds(g * _L, _L)
                z = jnp.maximum(z3_v[pj + j, sl], jnp.float32(1e-3))
                xt_v[j, sl] = x3_v[pj + j, sl] / z * fxv + cxv
                yt_v[j, sl] = y3_v[pj + j, sl] / z * fyv + cyv

        for gp in range(2):
            gA, gB = 2 * gp, 2 * gp + 1
            slA = pl.ds(gA * _L, _L)
            slB = pl.ds(gB * _L, _L)
            init = (jnp.full((_L,), jnp.float32(3.0e38)),
                    jnp.full((_L,), jnp.float32(3.0e38)),
                    jnp.zeros((_L,), jnp.int32),
                    jnp.zeros((_L,), jnp.int32))

            @plsc.parallel_loop(0, _NP, carry=init)
            def res(pr, carry, slA=slA, slB=slB):
                bdA, bdB, biA, biB = carry
                pr16 = jnp.full((_L,), pr, jnp.int32)
                base = pr16 * _NJP
                accA = jnp.zeros((_L,), jnp.float32)
                accB = jnp.zeros((_L,), jnp.float32)
                den = _spl(jnp.float32(1e-8))
                for j in range(_NJ):
                    idxj = base + j
                    xrs = plsc.load_gather(xr_v, [idxj])
                    yrs = plsc.load_gather(yr_v, [idxj])
                    vs = plsc.load_gather(vis_v, [idxj])
                    den = den + vs
                    dxA = xt_v[j, slA] - xrs
                    dyA = yt_v[j, slA] - yrs
                    accA = accA + vs * (dxA * dxA + dyA * dyA)
                    dxB = xt_v[j, slB] - xrs
                    dyB = yt_v[j, slB] - yrs
                    accB = accB + vs * (dxB * dxB + dyB * dyB)
                valid = pr16.astype(jnp.float32) < nprv
                dA = accA / den
                dA = jnp.where(valid, dA, _spl(jnp.float32(1e5)))
                takeA = dA < bdA
                biA = jnp.where(takeA, pr16, biA)
                bdA = jnp.where(takeA, dA, bdA)
                dB = accB / den
                dB = jnp.where(valid, dB, _spl(jnp.float32(1e5)))
                takeB = dB < bdB
                biB = jnp.where(takeB, pr16, biB)
                bdB = jnp.where(takeB, dB, bdB)
                return (bdA, bdB, biA, biB)

            bi_v[gA] = res[2]
            bi_v[gB] = res[3]

        @plsc.parallel_loop(0, _NG)
        def _g(g):
            sl = pl.ds(g * _L, _L)
            g17 = g * _NJ
            bibase = bi_v[g] * _NJP
            mv0 = None
            for j in range(_NJ):
                idxj = bibase + j
                mx = plsc.load_gather(xr_v, [idxj])
                my = plsc.load_gather(yr_v, [idxj])
                mv = plsc.load_gather(vis_v, [idxj])
                mx_v[g17 + j] = mx
                my_v[g17 + j] = my
                xtj = xt_v[j, sl]
                ytj = yt_v[j, sl]
                ddx = xtj - mx
                ddy = ytj - my
                md = ddx * ddx + ddy * ddy + jnp.float32(1e-12)
                score_v[pj + j, sl] = jnp.exp(_sqrt16(md) * jnp.float32(-0.02))
                inb = ((xtj >= jnp.float32(0.0)) & (ytj >= jnp.float32(0.0))
                       & (xtj <= jnp.float32(_IMW - 1))
                       & (ytj <= jnp.float32(_IMH - 1)))
                bound_v[pj + j, sl] = jnp.where(inb, mv, _spl(jnp.float32(0.0)))
                if j == 0:
                    mv0 = mv

            for k in range(_NB):
                a, c = _BONE_A[k], _BONE_B[k]
                ex = xt_v[a, sl] - xt_v[c, sl]
                ey = yt_v[a, sl] - yt_v[c, sl]
                blt = _sqrt16(ex * ex + ey * ey + jnp.float32(1e-12))
                exr = mx_v[g17 + a] - mx_v[g17 + c]
                eyr = my_v[g17 + a] - my_v[g17 + c]
                blr = _sqrt16(exr * exr + eyr * eyr + jnp.float32(1e-12))
                sbl_v[pk + k, sl] = jnp.exp(jnp.abs(blr - blt)
                                          * jnp.float32(-0.2))
                bound2_v[pk + k, sl] = mv0

        pltpu.async_copy(score_v.at[pl.ds(pj, _NJ)], score_hbm.at[b, pt], sem_out)
        pltpu.async_copy(sbl_v.at[pl.ds(pk, _NB)], sbl_hbm.at[b, pt], sem_out)
        pltpu.async_copy(bound_v.at[pl.ds(pj, _NJ)], bound_hbm.at[b, pt], sem_out)
        pltpu.async_copy(bound2_v.at[pl.ds(pk, _NB)], bound2_hbm.at[b, pt], sem_out)
        return 0

    lax.fori_loop(0, _NPT, pt_body, 0, unroll=False)

    for q in range(2):
        ptq = pt0 + _NPT - 2 + q
        pq = ptq % 2
        pltpu.make_async_copy(score_v.at[pl.ds(pq * _NJ, _NJ)], score_hbm.at[b, ptq],
                              sem_out).wait()
        pltpu.make_async_copy(sbl_v.at[pl.ds(pq * _NB, _NB)], sbl_hbm.at[b, ptq],
                              sem_out).wait()
        pltpu.make_async_copy(bound_v.at[pl.ds(pq * _NJ, _NJ)], bound_hbm.at[b, ptq],
                              sem_out).wait()
        pltpu.make_async_copy(bound2_v.at[pl.ds(pq * _NB, _NB)], bound2_hbm.at[b, ptq],
                              sem_out).wait()


@jax.jit
def kernel(poses_3d, poses_2d_ref, vis_ref, cam_f, cam_c, num_persons_ref):
    f32 = jnp.float32
    x3 = poses_3d[..., 0]
    y3 = poses_3d[..., 1]
    z3 = poses_3d[..., 2]
    pad = ((0, 0), (0, 0), (0, _NJP - _NJ))
    xr = jnp.pad(poses_2d_ref[..., 0], pad).reshape(_B, _NP * _NJP)
    yr = jnp.pad(poses_2d_ref[..., 1], pad).reshape(_B, _NP * _NJP)
    vis = jnp.pad(vis_ref, pad).reshape(_B, _NP * _NJP)
    cams = jnp.stack([cam_f[:, 0], cam_f[:, 1], cam_c[:, 0], cam_c[:, 1],
                      num_persons_ref.astype(f32),
                      jnp.zeros((_B,), f32), jnp.zeros((_B,), f32),
                      jnp.zeros((_B,), f32)], axis=1)
    cams16 = jnp.broadcast_to(cams[:, :, None], (_B, 8, _L)) + 0.0

    mesh = plsc.VectorSubcoreMesh(core_axis_name="c", subcore_axis_name="s",
                                  num_cores=2, num_subcores=16)
    out_type = [
        jax.ShapeDtypeStruct((_B, _NP, _NJ, _ND), f32),
        jax.ShapeDtypeStruct((_B, _NP, _NB, _ND), f32),
        jax.ShapeDtypeStruct((_B, _NP, _NJ, _ND), f32),
        jax.ShapeDtypeStruct((_B, _NP, _NB, _ND), f32),
    ]
    scratch = [
        pltpu.VMEM((8, _L), f32),
        pltpu.VMEM((_NP * _NJP,), f32),
        pltpu.VMEM((_NP * _NJP,), f32),
        pltpu.VMEM((_NP * _NJP,), f32),
        pltpu.VMEM((2 * _NJ, _ND), f32),
        pltpu.VMEM((2 * _NJ, _ND), f32),
        pltpu.VMEM((2 * _NJ, _ND), f32),
        pltpu.VMEM((_NJ, _ND), f32),
        pltpu.VMEM((_NJ, _ND), f32),
        pltpu.VMEM((_NG * _NJ, _L), f32),
        pltpu.VMEM((_NG * _NJ, _L), f32),
        pltpu.VMEM((_NG, _L), jnp.int32),
        pltpu.VMEM((2 * _NJ, _ND), f32),
        pltpu.VMEM((2 * _NB, _ND), f32),
        pltpu.VMEM((2 * _NJ, _ND), f32),
        pltpu.VMEM((2 * _NB, _ND), f32),
        pltpu.SemaphoreType.DMA,
        pltpu.SemaphoreType.DMA,
    ]
    outs = pl.kernel(
        _sc_body,
        out_type=out_type,
        mesh=mesh,
        scratch_types=scratch,
        compiler_params=pltpu.CompilerParams(needs_layout_passes=False),
    )(cams16, x3, y3, z3, xr, yr, vis)
    return tuple(outs)
